# baseline (speedup 1.0000x reference)
import functools

import jax
import jax.numpy as jnp
from jax import lax
from jax.experimental import pallas as pl
from jax.experimental.pallas import tpu as pltpu
from jax.experimental.pallas import tpu_sc as plsc

N = 10000
E = 320000
T = 4
F_IN = 128
K = 2

NP = 10240
DUMMY = N
NCORES = 2
NSUB = 16
NTILES = NCORES * NSUB
EPT = E // NTILES
CR = 125
C = 128
NCH = EPT // CR
ECH = E // CR
RPT = NP // NSUB
ZR = 128

DIN = [128, 256, 128]
DOUT = [256, 128, 2]
DP = [256, 128, 128]
XW = [[128], [128, 128], [128]]
HW = [[128, 128], [128], [128]]

BN = 1000
GRID = N // BN

_f32 = jnp.float32
_i32 = jnp.int32


def _mesh():
    return plsc.VectorSubcoreMesh(
        core_axis_name="c", subcore_axis_name="s",
        num_cores=NCORES, num_subcores=NSUB)


def _fill_rows(buf, nrows, w, vec16):
    def frow(r, carry):
        for j in range(w // 16):
            buf[r, pl.ds(j * 16, 16)] = vec16
        return carry

    lax.fori_loop(0, nrows, frow, 0)


@functools.lru_cache(maxsize=None)
def _make_prop(n_in):
    w = 128
    NH = NCH // 2
    scratch = (
        [pltpu.VMEM((NH, C), _i32),
         pltpu.VMEM((NH, C), _i32),
         pltpu.VMEM((2, C, w), _f32),
         pltpu.VMEM_SHARED((NP, w), _f32)]
        + [pltpu.SemaphoreType.DMA] * 2
    )
    out_type = [jax.ShapeDtypeStruct((NCORES, NP, w), _f32)
                for _ in range(n_in)]

    def body(*refs):
        row_h, col_h = refs[0], refs[1]
        xs = refs[2:2 + n_in]
        outs = refs[2 + n_in:2 + 2 * n_in]
        rbuf, cbuf, ring, accum = refs[2 + 2 * n_in:2 + 2 * n_in + 4]
        gsem = refs[2 + 2 * n_in + 4:]

        cid = lax.axis_index("c")
        sid = lax.axis_index("s")
        wid = cid * NSUB + sid
        rbase = sid * RPT

        def gstart(x_h, c, b):
            pltpu.async_copy(x_h.at[rbuf.at[c]], ring.at[b], gsem[b])

        def gwait(x_h, c, b):
            pltpu.make_async_copy(x_h.at[rbuf.at[c]], ring.at[b],
                                  gsem[b]).wait()

        for k in range(n_in):
            x_h = xs[k]
            out = outs[k]

            _fill_rows(ring.at[0], C, w, jnp.zeros((16,), _f32))

            def zcp(m, carry):
                pltpu.sync_copy(ring.at[0],
                                accum.at[pl.ds(rbase + m * C, C)])
                return carry

            lax.fori_loop(0, RPT // C, zcp, 0)
            plsc.subcore_barrier()

            for h in range(2):
                base = wid * NCH + h * NH
                pltpu.sync_copy(row_h.at[pl.ds(base, NH)], rbuf)
                pltpu.sync_copy(col_h.at[pl.ds(base, NH)], cbuf)

                gstart(x_h, 0, 0)
                gstart(x_h, 1, 1)
                for c in range(NH):
                    b = c % 2
                    gwait(x_h, c, b)
                    pltpu.sync_copy(ring.at[b], accum.at[cbuf.at[c]],
                                    add=True)
                    if c + 2 < NH:
                        gstart(x_h, c + 2, b)

            plsc.subcore_barrier()
            pltpu.sync_copy(accum.at[pl.ds(rbase, RPT)],
                            out.at[cid, pl.ds(rbase, RPT)])

    return pl.kernel(body, out_type=out_type, mesh=_mesh(),
                     scratch_types=scratch)


@functools.lru_cache(maxsize=None)
def _make_degree():
    w = 128
    scratch = [
        pltpu.VMEM((NCH, C), _i32),
        pltpu.VMEM((C, w), _f32),
        pltpu.VMEM_SHARED((NP, w), _f32),
    ]
    out_type = [jax.ShapeDtypeStruct((NCORES, NP, w), _f32)]

    def body(rowd_h, out, col_all, ones_buf, accum):
        cid = lax.axis_index("c")
        sid = lax.axis_index("s")
        wid = cid * NSUB + sid
        rbase = sid * RPT

        pltpu.sync_copy(rowd_h.at[pl.ds(wid * NCH, NCH)], col_all)
        _fill_rows(ones_buf, C, w, jnp.zeros((16,), _f32))

        def zcp(m, carry):
            pltpu.sync_copy(ones_buf, accum.at[pl.ds(rbase + m * C, C)])
            return carry

        lax.fori_loop(0, RPT // C, zcp, 0)
        _fill_rows(ones_buf, C, w, jnp.full((16,), 1.0, _f32))
        plsc.subcore_barrier()

        def chunk(i, carry):
            pltpu.sync_copy(ones_buf, accum.at[col_all.at[i]], add=True)
            return carry

        lax.fori_loop(0, NCH, chunk, 0)

        plsc.subcore_barrier()
        pltpu.sync_copy(accum.at[pl.ds(rbase, RPT)],
                        out.at[cid, pl.ds(rbase, RPT)])

    return pl.kernel(body, out_type=out_type, mesh=_mesh(),
                     scratch_types=scratch)


def _redirect(row, col):
    r2 = row.reshape(E // 128, 128)
    c2 = col.reshape(E // 128, 128)

    def body(r_ref, c_ref, col2_ref, rowd_ref):
        r = r_ref[...]
        c = c_ref[...]
        is_self = r == c
        col2_ref[...] = jnp.where(is_self, DUMMY, c)
        rowd_ref[...] = jnp.where(is_self, DUMMY, r)

    col2, rowd = pl.pallas_call(
        body,
        out_shape=[jax.ShapeDtypeStruct((E // 128, 128), _i32)] * 2,
    )(r2, c2)
    pad = C - CR
    row2d = jnp.pad(row.reshape(ECH, CR), ((0, 0), (0, pad)))
    col2d = jnp.pad(col2.reshape(ECH, CR), ((0, 0), (0, pad)),
                    constant_values=DUMMY)
    rowd2d = jnp.pad(rowd.reshape(ECH, CR), ((0, 0), (0, pad)),
                     constant_values=DUMMY)
    return row2d, col2d, rowd2d


def _prep(degp, graphs):
    def body(degp_ref, g_ref, dis_ref, gs_ref):
        deg = degp_ref[0, :, :16] + degp_ref[1, :, :16]
        dis = jnp.where(deg > 0, lax.rsqrt(deg), 0.0)
        dis_ref[...] = dis
        d1 = dis[:, :1]
        for t in range(T):
            gs_ref[t] = g_ref[t] * d1

    return pl.pallas_call(
        body,
        grid=(GRID,),
        in_specs=[
            pl.BlockSpec((NCORES, BN, 128), lambda i: (0, i, 0)),
            pl.BlockSpec((T, BN, F_IN), lambda i: (0, i, 0)),
        ],
        out_specs=[
            pl.BlockSpec((BN, 16), lambda i: (i, 0)),
            pl.BlockSpec((T, BN, F_IN), lambda i: (0, i, 0)),
        ],
        out_shape=[
            jax.ShapeDtypeStruct((N, 16), _f32),
            jax.ShapeDtypeStruct((T, N, F_IN), _f32),
        ],
    )(degp, graphs)


def _part_spec():
    return pl.BlockSpec((NCORES, BN, None), lambda i: (0, i, 0))


def _full(shape):
    nd = len(shape)
    return pl.BlockSpec(shape, lambda i: (0,) * nd)


def _gates(layer, t0, X, H, dis16, px_parts, ph_parts, wts):
    din = DIN[layer]
    dp = DP[layer]
    xw = XW[layer]
    hw = HW[layer]
    Wx0, Wx1, bx, Wh0, Wh1, bh, Whh0, bhh = wts[:8]
    nx = len(px_parts)
    nh = len(ph_parts)

    def body(*refs):
        i = 0
        x_ref = refs[i]; i += 1
        if not t0:
            h_ref = refs[i]; i += 1
        dis_ref = refs[i]; i += 1
        pxr = refs[i:i + nx]; i += nx
        phr = refs[i:i + nh] if not t0 else []
        i += len(phr)
        wx0_r, wx1_r, bx_r, wh0_r, wh1_r, bh_r, whh0_r, bhh_r = refs[i:i + 8]
        i += 8
        z_ref, a3_ref = refs[i], refs[i + 1]
        hrs_refs = refs[i + 2:]

        d1 = dis_ref[:, :1]
        nd = -d1
        ax = jnp.dot(x_ref[...], wx0_r[...], preferred_element_type=_f32)
        wx1 = wx1_r[...]
        off = 0
        for k, w in enumerate(xw):
            px = (pxr[k][0] + pxr[k][1]) * nd
            ax = ax + jnp.dot(px, wx1[off:off + w], preferred_element_type=_f32)
            off += w
        ax = ax + bx_r[...]

        if t0:
            ah = jnp.broadcast_to(bh_r[...], (BN, 2 * dp))
        else:
            h = h_ref[...]
            ah = jnp.dot(h, wh0_r[...], preferred_element_type=_f32)
            wh1 = wh1_r[...]
            off = 0
            for k, w in enumerate(hw):
                ph = (phr[k][0] + phr[k][1]) * nd
                ah = ah + jnp.dot(ph, wh1[off:off + w],
                                  preferred_element_type=_f32)
                off += w
            ah = ah + bh_r[...]

        z = jax.nn.sigmoid(ax[:, :dp] + ah[:, :dp])
        z_ref[...] = z
        if t0:
            a3_ref[...] = ax[:, 2 * dp:] + bhh_r[...]
        else:
            r = jax.nn.sigmoid(ax[:, dp:2 * dp] + ah[:, dp:2 * dp])
            hr = h * r
            a3_ref[...] = (ax[:, 2 * dp:] + bhh_r[...]
                           + jnp.dot(hr, whh0_r[...],
                                     preferred_element_type=_f32))
            off = 0
            for k, w in enumerate(hw):
                hrs_refs[k][...] = d1 * hr[:, off:off + w]
                off += w

    in_arrays = [X] + ([] if t0 else [H]) + [dis16] + list(px_parts) \
        + ([] if t0 else list(ph_parts)) + [Wx0, Wx1, bx, Wh0, Wh1, bh,
                                            Whh0, bhh]
    in_specs = [pl.BlockSpec((BN, din), lambda i: (i, 0))]
    if not t0:
        in_specs.append(pl.BlockSpec((BN, dp), lambda i: (i, 0)))
    in_specs.append(pl.BlockSpec((BN, 16), lambda i: (i, 0)))
    for w in xw:
        in_specs.append(pl.BlockSpec((NCORES, BN, w), lambda i: (0, i, 0)))
    if not t0:
        for w in hw:
            in_specs.append(pl.BlockSpec((NCORES, BN, w), lambda i: (0, i, 0)))
    for a in [Wx0, Wx1, bx, Wh0, Wh1, bh, Whh0, bhh]:
        in_specs.append(_full(a.shape))

    out_shape = [jax.ShapeDtypeStruct((N, dp), _f32),
                 jax.ShapeDtypeStruct((N, dp), _f32)]
    out_specs = [pl.BlockSpec((BN, dp), lambda i: (i, 0)),
                 pl.BlockSpec((BN, dp), lambda i: (i, 0))]
    if not t0:
        for w in hw:
            out_shape.append(jax.ShapeDtypeStruct((N, w), _f32))
            out_specs.append(pl.BlockSpec((BN, w), lambda i: (i, 0)))

    res = pl.pallas_call(
        body, grid=(GRID,), in_specs=in_specs,
        out_specs=out_specs, out_shape=out_shape,
    )(*in_arrays)
    if t0:
        return res[0], res[1], []
    return res[0], res[1], list(res[2:])


def _final(layer, t0, Z, A3p, H, dis16, phr_parts, Whh1):
    dp = DP[layer]
    hw = HW[layer]
    relu = layer > 0
    nh = len(phr_parts)

    def body(*refs):
        i = 0
        z_ref = refs[i]; i += 1
        a3_ref = refs[i]; i += 1
        if not t0:
            h_ref = refs[i]; i += 1
        dis_ref = refs[i]; i += 1
        phr = refs[i:i + nh]; i += nh
        if not t0:
            whh1_r = refs[i]; i += 1
        hn_ref = refs[i]
        hns_refs = refs[i + 1:]

        d1 = dis_ref[:, :1]
        ht_in = a3_ref[...]
        if not t0:
            whh1 = whh1_r[...]
            nd = -d1
            off = 0
            for k, w in enumerate(hw):
                p = (phr[k][0] + phr[k][1]) * nd
                ht_in = ht_in + jnp.dot(p, whh1[off:off + w],
                                        preferred_element_type=_f32)
                off += w
        ht = jnp.tanh(ht_in)
        z = z_ref[...]
        if t0:
            hn = (1.0 - z) * ht
        else:
            hn = z * h_ref[...] + (1.0 - z) * ht
        if relu:
            hn = jnp.maximum(hn, 0.0)
        hn_ref[...] = hn
        off = 0
        for k, w in enumerate(hw):
            hns_refs[k][...] = d1 * hn[:, off:off + w]
            off += w

    in_arrays = [Z, A3p] + ([] if t0 else [H]) + [dis16]
    in_specs = [pl.BlockSpec((BN, dp), lambda i: (i, 0)),
                pl.BlockSpec((BN, dp), lambda i: (i, 0))]
    if not t0:
        in_specs.append(pl.BlockSpec((BN, dp), lambda i: (i, 0)))
    in_specs.append(pl.BlockSpec((BN, 16), lambda i: (i, 0)))
    if not t0:
        in_arrays += list(phr_parts)
        for w in hw:
            in_specs.append(pl.BlockSpec((NCORES, BN, w), lambda i: (0, i, 0)))
        in_arrays.append(Whh1)
        in_specs.append(_full(Whh1.shape))

    out_shape = [jax.ShapeDtypeStruct((N, dp), _f32)]
    out_specs = [pl.BlockSpec((BN, dp), lambda i: (i, 0))]
    for w in hw:
        out_shape.append(jax.ShapeDtypeStruct((N, w), _f32))
        out_specs.append(pl.BlockSpec((BN, w), lambda i: (i, 0)))

    res = pl.pallas_call(
        body, grid=(GRID,), in_specs=in_specs,
        out_specs=out_specs, out_shape=out_shape,
    )(*in_arrays)
    return res[0], list(res[1:])


def _prep_weights(params):
    wts = []
    for layer, lp in enumerate(params):
        dout = DOUT[layer]
        dp = DP[layer]
        cpad = dp - dout

        def padw(w, rpad):
            return jnp.pad(w, ((0, rpad), (0, cpad)))

        Wx0 = jnp.concatenate(
            [padw(lp[g]['W'][0], 0) for g in ('x_z', 'x_r', 'x_h')], axis=1)
        Wx1 = jnp.concatenate(
            [padw(lp[g]['W'][1], 0) for g in ('x_z', 'x_r', 'x_h')], axis=1)
        bx = jnp.concatenate(
            [jnp.pad(lp[g]['b'], (0, cpad)) for g in ('x_z', 'x_r', 'x_h')]
        ).reshape(1, 3 * dp)
        Wh0 = jnp.concatenate(
            [padw(lp[g]['W'][0], cpad) for g in ('h_z', 'h_r')], axis=1)
        Wh1 = jnp.concatenate(
            [padw(lp[g]['W'][1], cpad) for g in ('h_z', 'h_r')], axis=1)
        bh = jnp.concatenate(
            [jnp.pad(lp[g]['b'], (0, cpad)) for g in ('h_z', 'h_r')]
        ).reshape(1, 2 * dp)
        Whh0 = padw(lp['h_h']['W'][0], cpad)
        Whh1 = padw(lp['h_h']['W'][1], cpad)
        bhh = jnp.pad(lp['h_h']['b'], (0, cpad)).reshape(1, dp)
        wts.append((Wx0, Wx1, bx, Wh0, Wh1, bh, Whh0, bhh, Whh1))
    return wts


def kernel(graphs, edge_index, params):
    row = edge_index[0]
    col = edge_index[1]
    row2d, col2d, rowd2d = _redirect(row, col)

    (degp,) = _make_degree()(rowd2d)
    dis16, gs = _prep(degp, graphs)

    wts = _prep_weights(params)

    def prop(chunks):
        return list(_make_prop(len(chunks))(row2d, col2d, *chunks))

    H = [None] * 3
    Hs = [None] * 3
    ph_cache = [None] * 3
    preds = []
    for t in range(T):
        t0 = t == 0
        for i in range(3):
            if i == 0:
                x = graphs[t]
                xs_chunks = [gs[t]]
            else:
                x = H[i - 1]
                xs_chunks = Hs[i - 1]
            nx = len(xs_chunks)
            if i == 2 and not t0:
                parts = prop(list(xs_chunks) + list(Hs[2]))
                px = parts[:nx]
                ph_cache[2] = parts[nx:]
            else:
                px = prop(list(xs_chunks))
            ph = [] if t0 else ph_cache[i]
            if i >= 1:
                ph_cache[i - 1] = px
            z, a3p, hrs = _gates(i, t0, x, H[i], dis16, px, ph, wts[i])
            if t0:
                phr = []
            else:
                phr = prop(hrs)
            hn, hns = _final(i, t0, z, a3p, H[i], dis16, phr, wts[i][8])
            H[i] = hn
            Hs[i] = hns
        preds.append(H[2][:, :2])
    return jnp.stack(preds)


# speedup vs baseline: 1.4282x; 1.4282x over previous
import functools

import jax
import jax.numpy as jnp
from jax import lax
from jax.experimental import pallas as pl
from jax.experimental.pallas import tpu as pltpu
from jax.experimental.pallas import tpu_sc as plsc

N = 10000
E = 320000
T = 4
F_IN = 128
K = 2

NP = 10240
DUMMY = N
NCORES = 2
NSUB = 16
NTILES = NCORES * NSUB
EPT = E // NTILES
CR = 125
C = 128
NCH = EPT // CR
ECH = E // CR
RPT = NP // NSUB
ZR = 128

DIN = [128, 256, 128]
DOUT = [256, 128, 2]
DP = [256, 128, 128]
XW = [[128], [128, 128], [128]]
HW = [[128, 128], [128], [128]]

BN = 1000
GRID = N // BN

_f32 = jnp.float32
_i32 = jnp.int32


def _mesh():
    return plsc.VectorSubcoreMesh(
        core_axis_name="c", subcore_axis_name="s",
        num_cores=NCORES, num_subcores=NSUB)


def _fill_rows(buf, nrows, w, vec16):
    def frow(r, carry):
        for j in range(w // 16):
            buf[r, pl.ds(j * 16, 16)] = vec16
        return carry

    lax.fori_loop(0, nrows, frow, 0)


@functools.lru_cache(maxsize=None)
def _make_prop(n_in):
    w = 128
    CS = 80
    NCS = EPT // CS
    scratch = (
        [pltpu.VMEM((2, CS), _i32),
         pltpu.VMEM((2, CS), _i32),
         pltpu.VMEM((2, CS, w), _f32),
         pltpu.VMEM_SHARED((NP, w), _f32)]
        + [pltpu.SemaphoreType.DMA] * 2
    )
    out_type = [jax.ShapeDtypeStruct((NCORES, NP, w), _f32)
                for _ in range(n_in)]

    def body(*refs):
        row_h, col_h = refs[0], refs[1]
        xs = refs[2:2 + n_in]
        outs = refs[2 + n_in:2 + 2 * n_in]
        rbuf, cbuf, ring, accum = refs[2 + 2 * n_in:2 + 2 * n_in + 4]
        gsem = refs[2 + 2 * n_in + 4:]

        cid = lax.axis_index("c")
        sid = lax.axis_index("s")
        ebase = (cid * NSUB + sid) * EPT
        rbase = sid * RPT

        def gstart(x_h, b):
            pltpu.async_copy(x_h.at[rbuf.at[b]], ring.at[b], gsem[b])

        def gwait(x_h, b):
            pltpu.make_async_copy(x_h.at[rbuf.at[b]], ring.at[b],
                                  gsem[b]).wait()

        for k in range(n_in):
            x_h = xs[k]
            out = outs[k]

            _fill_rows(ring.at[0], CS, w, jnp.zeros((16,), _f32))

            def zcp(m, carry):
                pltpu.sync_copy(ring.at[0],
                                accum.at[pl.ds(rbase + m * CS, CS)])
                return carry

            lax.fori_loop(0, RPT // CS, zcp, 0)
            plsc.subcore_barrier()

            def pair(i, carry):
                for b in (0, 1):
                    off = ebase + (i * 2 + b) * CS
                    pltpu.sync_copy(row_h.at[pl.ds(off, CS)], rbuf.at[b])
                    pltpu.sync_copy(col_h.at[pl.ds(off, CS)], cbuf.at[b])
                gstart(x_h, 0)
                gstart(x_h, 1)
                for b in (0, 1):
                    gwait(x_h, b)
                    pltpu.sync_copy(ring.at[b], accum.at[cbuf.at[b]],
                                    add=True)
                return carry

            lax.fori_loop(0, NCS // 2, pair, 0)
            off = ebase + (NCS - 1) * CS
            pltpu.sync_copy(row_h.at[pl.ds(off, CS)], rbuf.at[0])
            pltpu.sync_copy(col_h.at[pl.ds(off, CS)], cbuf.at[0])
            gstart(x_h, 0)
            gwait(x_h, 0)
            pltpu.sync_copy(ring.at[0], accum.at[cbuf.at[0]], add=True)

            plsc.subcore_barrier()
            pltpu.sync_copy(accum.at[pl.ds(rbase, RPT)],
                            out.at[cid, pl.ds(rbase, RPT)])

    return pl.kernel(body, out_type=out_type, mesh=_mesh(),
                     scratch_types=scratch)


@functools.lru_cache(maxsize=None)
def _make_degree():
    w = 128
    scratch = [
        pltpu.VMEM((NCH, C), _i32),
        pltpu.VMEM((C, w), _f32),
        pltpu.VMEM_SHARED((NP, w), _f32),
    ]
    out_type = [jax.ShapeDtypeStruct((NCORES, NP, w), _f32)]

    def body(rowd_h, out, col_all, ones_buf, accum):
        cid = lax.axis_index("c")
        sid = lax.axis_index("s")
        wid = cid * NSUB + sid
        rbase = sid * RPT

        pltpu.sync_copy(rowd_h.at[pl.ds(wid * NCH, NCH)], col_all)
        _fill_rows(ones_buf, C, w, jnp.zeros((16,), _f32))

        def zcp(m, carry):
            pltpu.sync_copy(ones_buf, accum.at[pl.ds(rbase + m * C, C)])
            return carry

        lax.fori_loop(0, RPT // C, zcp, 0)
        _fill_rows(ones_buf, C, w, jnp.full((16,), 1.0, _f32))
        plsc.subcore_barrier()

        def chunk(i, carry):
            pltpu.sync_copy(ones_buf, accum.at[col_all.at[i]], add=True)
            return carry

        lax.fori_loop(0, NCH, chunk, 0)

        plsc.subcore_barrier()
        pltpu.sync_copy(accum.at[pl.ds(rbase, RPT)],
                        out.at[cid, pl.ds(rbase, RPT)])

    return pl.kernel(body, out_type=out_type, mesh=_mesh(),
                     scratch_types=scratch)


def _redirect(row, col):
    r2 = row.reshape(E // 128, 128)
    c2 = col.reshape(E // 128, 128)

    def body(r_ref, c_ref, col2_ref, rowd_ref):
        r = r_ref[...]
        c = c_ref[...]
        is_self = r == c
        col2_ref[...] = jnp.where(is_self, DUMMY, c)
        rowd_ref[...] = jnp.where(is_self, DUMMY, r)

    col2, rowd = pl.pallas_call(
        body,
        out_shape=[jax.ShapeDtypeStruct((E // 128, 128), _i32)] * 2,
    )(r2, c2)
    rowd2d = jnp.pad(rowd.reshape(ECH, CR), ((0, 0), (0, C - CR)),
                     constant_values=DUMMY)
    return col2.reshape(E), rowd2d


def _prep(degp, graphs):
    def body(degp_ref, g_ref, dis_ref, gs_ref):
        deg = degp_ref[0, :, :16] + degp_ref[1, :, :16]
        dis = jnp.where(deg > 0, lax.rsqrt(deg), 0.0)
        dis_ref[...] = dis
        d1 = dis[:, :1]
        for t in range(T):
            gs_ref[t] = g_ref[t] * d1

    return pl.pallas_call(
        body,
        grid=(GRID,),
        in_specs=[
            pl.BlockSpec((NCORES, BN, 128), lambda i: (0, i, 0)),
            pl.BlockSpec((T, BN, F_IN), lambda i: (0, i, 0)),
        ],
        out_specs=[
            pl.BlockSpec((BN, 16), lambda i: (i, 0)),
            pl.BlockSpec((T, BN, F_IN), lambda i: (0, i, 0)),
        ],
        out_shape=[
            jax.ShapeDtypeStruct((N, 16), _f32),
            jax.ShapeDtypeStruct((T, N, F_IN), _f32),
        ],
    )(degp, graphs)


def _part_spec():
    return pl.BlockSpec((NCORES, BN, None), lambda i: (0, i, 0))


def _full(shape):
    nd = len(shape)
    return pl.BlockSpec(shape, lambda i: (0,) * nd)


def _gates(layer, t0, X, H, dis16, px_parts, ph_parts, wts):
    din = DIN[layer]
    dp = DP[layer]
    xw = XW[layer]
    hw = HW[layer]
    Wx0, Wx1, bx, Wh0, Wh1, bh, Whh0, bhh = wts[:8]
    nx = len(px_parts)
    nh = len(ph_parts)

    def body(*refs):
        i = 0
        x_ref = refs[i]; i += 1
        if not t0:
            h_ref = refs[i]; i += 1
        dis_ref = refs[i]; i += 1
        pxr = refs[i:i + nx]; i += nx
        phr = refs[i:i + nh] if not t0 else []
        i += len(phr)
        wx0_r, wx1_r, bx_r, wh0_r, wh1_r, bh_r, whh0_r, bhh_r = refs[i:i + 8]
        i += 8
        z_ref, a3_ref = refs[i], refs[i + 1]
        hrs_refs = refs[i + 2:]

        d1 = dis_ref[:, :1]
        nd = -d1
        ax = jnp.dot(x_ref[...], wx0_r[...], preferred_element_type=_f32)
        wx1 = wx1_r[...]
        off = 0
        for k, w in enumerate(xw):
            px = (pxr[k][0] + pxr[k][1]) * nd
            ax = ax + jnp.dot(px, wx1[off:off + w], preferred_element_type=_f32)
            off += w
        ax = ax + bx_r[...]

        if t0:
            ah = jnp.broadcast_to(bh_r[...], (BN, 2 * dp))
        else:
            h = h_ref[...]
            ah = jnp.dot(h, wh0_r[...], preferred_element_type=_f32)
            wh1 = wh1_r[...]
            off = 0
            for k, w in enumerate(hw):
                ph = (phr[k][0] + phr[k][1]) * nd
                ah = ah + jnp.dot(ph, wh1[off:off + w],
                                  preferred_element_type=_f32)
                off += w
            ah = ah + bh_r[...]

        z = jax.nn.sigmoid(ax[:, :dp] + ah[:, :dp])
        z_ref[...] = z
        if t0:
            a3_ref[...] = ax[:, 2 * dp:] + bhh_r[...]
        else:
            r = jax.nn.sigmoid(ax[:, dp:2 * dp] + ah[:, dp:2 * dp])
            hr = h * r
            a3_ref[...] = (ax[:, 2 * dp:] + bhh_r[...]
                           + jnp.dot(hr, whh0_r[...],
                                     preferred_element_type=_f32))
            off = 0
            for k, w in enumerate(hw):
                hrs_refs[k][...] = d1 * hr[:, off:off + w]
                off += w

    in_arrays = [X] + ([] if t0 else [H]) + [dis16] + list(px_parts) \
        + ([] if t0 else list(ph_parts)) + [Wx0, Wx1, bx, Wh0, Wh1, bh,
                                            Whh0, bhh]
    in_specs = [pl.BlockSpec((BN, din), lambda i: (i, 0))]
    if not t0:
        in_specs.append(pl.BlockSpec((BN, dp), lambda i: (i, 0)))
    in_specs.append(pl.BlockSpec((BN, 16), lambda i: (i, 0)))
    for w in xw:
        in_specs.append(pl.BlockSpec((NCORES, BN, w), lambda i: (0, i, 0)))
    if not t0:
        for w in hw:
            in_specs.append(pl.BlockSpec((NCORES, BN, w), lambda i: (0, i, 0)))
    for a in [Wx0, Wx1, bx, Wh0, Wh1, bh, Whh0, bhh]:
        in_specs.append(_full(a.shape))

    out_shape = [jax.ShapeDtypeStruct((N, dp), _f32),
                 jax.ShapeDtypeStruct((N, dp), _f32)]
    out_specs = [pl.BlockSpec((BN, dp), lambda i: (i, 0)),
                 pl.BlockSpec((BN, dp), lambda i: (i, 0))]
    if not t0:
        for w in hw:
            out_shape.append(jax.ShapeDtypeStruct((N, w), _f32))
            out_specs.append(pl.BlockSpec((BN, w), lambda i: (i, 0)))

    res = pl.pallas_call(
        body, grid=(GRID,), in_specs=in_specs,
        out_specs=out_specs, out_shape=out_shape,
    )(*in_arrays)
    if t0:
        return res[0], res[1], []
    return res[0], res[1], list(res[2:])


def _final(layer, t0, Z, A3p, H, dis16, phr_parts, Whh1):
    dp = DP[layer]
    hw = HW[layer]
    relu = layer > 0
    nh = len(phr_parts)

    def body(*refs):
        i = 0
        z_ref = refs[i]; i += 1
        a3_ref = refs[i]; i += 1
        if not t0:
            h_ref = refs[i]; i += 1
        dis_ref = refs[i]; i += 1
        phr = refs[i:i + nh]; i += nh
        if not t0:
            whh1_r = refs[i]; i += 1
        hn_ref = refs[i]
        hns_refs = refs[i + 1:]

        d1 = dis_ref[:, :1]
        ht_in = a3_ref[...]
        if not t0:
            whh1 = whh1_r[...]
            nd = -d1
            off = 0
            for k, w in enumerate(hw):
                p = (phr[k][0] + phr[k][1]) * nd
                ht_in = ht_in + jnp.dot(p, whh1[off:off + w],
                                        preferred_element_type=_f32)
                off += w
        ht = jnp.tanh(ht_in)
        z = z_ref[...]
        if t0:
            hn = (1.0 - z) * ht
        else:
            hn = z * h_ref[...] + (1.0 - z) * ht
        if relu:
            hn = jnp.maximum(hn, 0.0)
        hn_ref[...] = hn
        off = 0
        for k, w in enumerate(hw):
            hns_refs[k][...] = d1 * hn[:, off:off + w]
            off += w

    in_arrays = [Z, A3p] + ([] if t0 else [H]) + [dis16]
    in_specs = [pl.BlockSpec((BN, dp), lambda i: (i, 0)),
                pl.BlockSpec((BN, dp), lambda i: (i, 0))]
    if not t0:
        in_specs.append(pl.BlockSpec((BN, dp), lambda i: (i, 0)))
    in_specs.append(pl.BlockSpec((BN, 16), lambda i: (i, 0)))
    if not t0:
        in_arrays += list(phr_parts)
        for w in hw:
            in_specs.append(pl.BlockSpec((NCORES, BN, w), lambda i: (0, i, 0)))
        in_arrays.append(Whh1)
        in_specs.append(_full(Whh1.shape))

    out_shape = [jax.ShapeDtypeStruct((N, dp), _f32)]
    out_specs = [pl.BlockSpec((BN, dp), lambda i: (i, 0))]
    for w in hw:
        out_shape.append(jax.ShapeDtypeStruct((N, w), _f32))
        out_specs.append(pl.BlockSpec((BN, w), lambda i: (i, 0)))

    res = pl.pallas_call(
        body, grid=(GRID,), in_specs=in_specs,
        out_specs=out_specs, out_shape=out_shape,
    )(*in_arrays)
    return res[0], list(res[1:])


def _prep_weights(params):
    wts = []
    for layer, lp in enumerate(params):
        dout = DOUT[layer]
        dp = DP[layer]
        cpad = dp - dout

        def padw(w, rpad):
            return jnp.pad(w, ((0, rpad), (0, cpad)))

        Wx0 = jnp.concatenate(
            [padw(lp[g]['W'][0], 0) for g in ('x_z', 'x_r', 'x_h')], axis=1)
        Wx1 = jnp.concatenate(
            [padw(lp[g]['W'][1], 0) for g in ('x_z', 'x_r', 'x_h')], axis=1)
        bx = jnp.concatenate(
            [jnp.pad(lp[g]['b'], (0, cpad)) for g in ('x_z', 'x_r', 'x_h')]
        ).reshape(1, 3 * dp)
        Wh0 = jnp.concatenate(
            [padw(lp[g]['W'][0], cpad) for g in ('h_z', 'h_r')], axis=1)
        Wh1 = jnp.concatenate(
            [padw(lp[g]['W'][1], cpad) for g in ('h_z', 'h_r')], axis=1)
        bh = jnp.concatenate(
            [jnp.pad(lp[g]['b'], (0, cpad)) for g in ('h_z', 'h_r')]
        ).reshape(1, 2 * dp)
        Whh0 = padw(lp['h_h']['W'][0], cpad)
        Whh1 = padw(lp['h_h']['W'][1], cpad)
        bhh = jnp.pad(lp['h_h']['b'], (0, cpad)).reshape(1, dp)
        wts.append((Wx0, Wx1, bx, Wh0, Wh1, bh, Whh0, bhh, Whh1))
    return wts


def kernel(graphs, edge_index, params):
    row = edge_index[0]
    col = edge_index[1]
    col2, rowd2d = _redirect(row, col)

    (degp,) = _make_degree()(rowd2d)
    dis16, gs = _prep(degp, graphs)

    wts = _prep_weights(params)

    def prop(chunks):
        return list(_make_prop(len(chunks))(row, col2, *chunks))

    H = [None] * 3
    Hs = [None] * 3
    ph_cache = [None] * 3
    preds = []
    for t in range(T):
        t0 = t == 0
        for i in range(3):
            if i == 0:
                x = graphs[t]
                xs_chunks = [gs[t]]
            else:
                x = H[i - 1]
                xs_chunks = Hs[i - 1]
            nx = len(xs_chunks)
            if i == 2 and not t0:
                parts = prop(list(xs_chunks) + list(Hs[2]))
                px = parts[:nx]
                ph_cache[2] = parts[nx:]
            else:
                px = prop(list(xs_chunks))
            ph = [] if t0 else ph_cache[i]
            if i >= 1:
                ph_cache[i - 1] = px
            z, a3p, hrs = _gates(i, t0, x, H[i], dis16, px, ph, wts[i])
            if t0:
                phr = []
            else:
                phr = prop(hrs)
            hn, hns = _final(i, t0, z, a3p, H[i], dis16, phr, wts[i][8])
            H[i] = hn
            Hs[i] = hns
        preds.append(H[2][:, :2])
    return jnp.stack(preds)
